# baseline (speedup 1.0000x reference)
import functools

import jax
import jax.numpy as jnp
from jax.experimental import pallas as pl

K = 3
EPS = 1e-8
CLAMP = 6.0
QB = 256


def _knn_block(q_ref, st_ref, s2_ref, feats_ref, lnw_ref, lnb_ref, out_ref):
    qb = q_ref[...]
    st = st_ref[...]
    n = st.shape[1]

    qs = jnp.dot(qb, st, preferred_element_type=jnp.float32)
    q2 = qb[:, 4:5]
    dsel = jnp.maximum((q2 + s2_ref[...]) + qs, 0.0)

    cols = jax.lax.broadcasted_iota(jnp.int32, dsel.shape, 1)
    inf = jnp.float32(jnp.inf)

    dwork = dsel
    dks = []
    jks = []
    for k in range(K):
        dk = jnp.min(dwork, axis=1, keepdims=True)
        jk = jnp.min(jnp.where(dwork == dk, cols, n), axis=1, keepdims=True)
        dks.append(jnp.sqrt(dk))
        jks.append(jk)
        if k < K - 1:
            dwork = jnp.where(cols == jk, inf, dwork)

    d1, d2k, d3 = dks
    scale = jnp.maximum((d1 + d2k + d3) * (1.0 / K), EPS)
    e1 = jnp.ones_like(d1)
    e2 = jnp.exp(-(d2k - d1) / scale)
    e3 = jnp.exp(-(d3 - d1) / scale)
    inv = 1.0 / (e1 + e2 + e3)
    w1, w2, w3 = e1 * inv, e2 * inv, e3 * inv

    zero = jnp.zeros_like(dsel)
    a = jnp.where(cols == jks[0], w1, zero)
    a = jnp.where(cols == jks[1], w2, a)
    a = jnp.where(cols == jks[2], w3, a)

    fi = jnp.dot(a, feats_ref[...], preferred_element_type=jnp.float32)

    x = fi + fi
    mu = jnp.mean(x, axis=1, keepdims=True)
    var = jnp.mean(x * x, axis=1, keepdims=True) - mu * mu
    y = (x - mu) * jax.lax.rsqrt(var + 1e-5)
    y = y * lnw_ref[...] + lnb_ref[...]
    out_ref[...] = jnp.tanh(y) * CLAMP


@functools.partial(jax.jit, static_argnames=("interpret",))
def kernel(coords, feats, gt_coords, ln_weight, ln_bias, interpret=False):
    n = coords.shape[0]
    m = gt_coords.shape[0]
    c = feats.shape[1]

    q = gt_coords.astype(jnp.float32)
    s = coords.astype(jnp.float32)
    all_cs = jnp.concatenate([q, s], axis=0)
    mu = all_cs.mean(axis=0)
    sd = all_cs.std(axis=0, ddof=1)
    sd = jnp.where(jnp.abs(sd) < EPS, 1.0, sd)
    q = (q - mu) / sd
    s = (s - mu) / sd

    q2 = jnp.sum(q * q, axis=1, keepdims=True)
    s2 = jnp.sum(s * s, axis=1, keepdims=True)
    zq = jnp.zeros((m, 4), jnp.float32)
    qp = jnp.concatenate([-2.0 * q, zq[:, :1], q2, zq[:, :3]], axis=1)
    stp = jnp.pad(s, ((0, 0), (0, 5))).T

    grid = m // QB
    fo = pl.pallas_call(
        _knn_block,
        grid=(grid,),
        in_specs=[
            pl.BlockSpec((QB, 8), lambda i: (i, 0)),
            pl.BlockSpec((8, n), lambda i: (0, 0)),
            pl.BlockSpec((1, n), lambda i: (0, 0)),
            pl.BlockSpec((n, c), lambda i: (0, 0)),
            pl.BlockSpec((1, c), lambda i: (0, 0)),
            pl.BlockSpec((1, c), lambda i: (0, 0)),
        ],
        out_specs=pl.BlockSpec((QB, c), lambda i: (i, 0)),
        out_shape=jax.ShapeDtypeStruct((m, c), jnp.float32),
        interpret=interpret,
    )(qp, stp, s2.T, feats, ln_weight.reshape(1, c), ln_bias.reshape(1, c))

    return (gt_coords, fo)


# speedup vs baseline: 1.0528x; 1.0528x over previous
import functools

import jax
import jax.numpy as jnp
from jax.experimental import pallas as pl

K = 3
EPS = 1e-8
CLAMP = 6.0
QB = 512


def _knn_block(q_ref, st_ref, s2_ref, feats_ref, lnw_ref, lnb_ref, out_ref):
    qb = q_ref[...]
    st = st_ref[...]
    n = st.shape[1]

    qs = jnp.dot(qb, st, preferred_element_type=jnp.float32)
    q2 = qb[:, 4:5]
    dsel = jnp.maximum((q2 + s2_ref[...]) + qs, 0.0)

    cols = jax.lax.broadcasted_iota(jnp.int32, dsel.shape, 1)
    inf = jnp.float32(jnp.inf)

    dwork = dsel
    dks = []
    jks = []
    for k in range(K):
        dk = jnp.min(dwork, axis=1, keepdims=True)
        jk = jnp.min(jnp.where(dwork == dk, cols, n), axis=1, keepdims=True)
        dks.append(jnp.sqrt(dk))
        jks.append(jk)
        if k < K - 1:
            dwork = jnp.where(cols == jk, inf, dwork)

    d1, d2k, d3 = dks
    scale = jnp.maximum((d1 + d2k + d3) * (1.0 / K), EPS)
    e1 = jnp.ones_like(d1)
    e2 = jnp.exp(-(d2k - d1) / scale)
    e3 = jnp.exp(-(d3 - d1) / scale)
    inv = 1.0 / (e1 + e2 + e3)
    w1, w2, w3 = e1 * inv, e2 * inv, e3 * inv

    zero = jnp.zeros_like(dsel)
    a = jnp.where(cols == jks[0], w1, zero)
    a = jnp.where(cols == jks[1], w2, a)
    a = jnp.where(cols == jks[2], w3, a)

    fi = jnp.dot(a, feats_ref[...], preferred_element_type=jnp.float32)

    x = fi + fi
    mu = jnp.mean(x, axis=1, keepdims=True)
    var = jnp.mean(x * x, axis=1, keepdims=True) - mu * mu
    y = (x - mu) * jax.lax.rsqrt(var + 1e-5)
    y = y * lnw_ref[...] + lnb_ref[...]
    out_ref[...] = jnp.tanh(y) * CLAMP


@functools.partial(jax.jit, static_argnames=("interpret",))
def kernel(coords, feats, gt_coords, ln_weight, ln_bias, interpret=False):
    n = coords.shape[0]
    m = gt_coords.shape[0]
    c = feats.shape[1]

    q = gt_coords.astype(jnp.float32)
    s = coords.astype(jnp.float32)
    all_cs = jnp.concatenate([q, s], axis=0)
    mu = all_cs.mean(axis=0)
    sd = all_cs.std(axis=0, ddof=1)
    sd = jnp.where(jnp.abs(sd) < EPS, 1.0, sd)
    q = (q - mu) / sd
    s = (s - mu) / sd

    q2 = jnp.sum(q * q, axis=1, keepdims=True)
    s2 = jnp.sum(s * s, axis=1, keepdims=True)
    zq = jnp.zeros((m, 4), jnp.float32)
    qp = jnp.concatenate([-2.0 * q, zq[:, :1], q2, zq[:, :3]], axis=1)
    stp = jnp.pad(s, ((0, 0), (0, 5))).T

    grid = m // QB
    fo = pl.pallas_call(
        _knn_block,
        grid=(grid,),
        in_specs=[
            pl.BlockSpec((QB, 8), lambda i: (i, 0)),
            pl.BlockSpec((8, n), lambda i: (0, 0)),
            pl.BlockSpec((1, n), lambda i: (0, 0)),
            pl.BlockSpec((n, c), lambda i: (0, 0)),
            pl.BlockSpec((1, c), lambda i: (0, 0)),
            pl.BlockSpec((1, c), lambda i: (0, 0)),
        ],
        out_specs=pl.BlockSpec((QB, c), lambda i: (i, 0)),
        out_shape=jax.ShapeDtypeStruct((m, c), jnp.float32),
        interpret=interpret,
    )(qp, stp, s2.T, feats, ln_weight.reshape(1, c), ln_bias.reshape(1, c))

    return (gt_coords, fo)


# speedup vs baseline: 1.1402x; 1.0830x over previous
import functools

import jax
import jax.numpy as jnp
from jax.experimental import pallas as pl

K = 3
EPS = 1e-8
CLAMP = 6.0
QB = 512


def _knn_block(q_ref, st_ref, s2_ref, ramp_ref, feats_ref, lnw_ref, lnb_ref, out_ref):
    qb = q_ref[...]
    st = st_ref[...]
    n = st.shape[1]

    qs = jnp.dot(qb, st, preferred_element_type=jnp.float32)
    q2 = qb[:, 4:5]
    dsel = jnp.maximum((q2 + s2_ref[...]) + qs, 0.0)
    dsel = jnp.where(dsel == 0.0, ramp_ref[...], dsel)
    inf = jnp.float32(jnp.inf)
    one = jnp.float32(1.0)

    v1 = jnp.min(dsel, axis=1, keepdims=True)
    eq1 = dsel == v1
    c1 = jnp.sum(jnp.where(eq1, one, 0.0), axis=1, keepdims=True)
    dw2 = jnp.where(eq1, inf, dsel)
    v2 = jnp.min(dw2, axis=1, keepdims=True)
    eq2 = dw2 == v2
    c2 = jnp.sum(jnp.where(eq2, one, 0.0), axis=1, keepdims=True)
    dw3 = jnp.where(eq2, inf, dw2)
    v3 = jnp.min(dw3, axis=1, keepdims=True)
    cols = jax.lax.broadcasted_iota(jnp.int32, (1, n), 1)
    j3 = jnp.min(jnp.where(dw3 == v3, cols, n), axis=1, keepdims=True)

    v2c = jnp.where(c1 >= 2.0, v1, v2)
    v3c = jnp.where(c1 >= 3.0, v1,
                    jnp.where(c1 >= 2.0, v2, jnp.where(c2 >= 2.0, v2, v3)))

    d1 = jnp.sqrt(jnp.where(v1 < 1e-20, 0.0, v1))
    d2k = jnp.sqrt(jnp.where(v2c < 1e-20, 0.0, v2c))
    d3 = jnp.sqrt(jnp.where(v3c < 1e-20, 0.0, v3c))
    scale = jnp.maximum((d1 + d2k + d3) * (1.0 / K), EPS)
    e1 = jnp.ones_like(d1)
    e2 = jnp.exp(-(d2k - d1) / scale)
    e3 = jnp.exp(-(d3 - d1) / scale)
    inv = 1.0 / (e1 + e2 + e3)
    w1, w2, w3 = e1 * inv, e2 * inv, e3 * inv

    wv2 = jnp.where(c1 >= 3.0, 0.0, jnp.where(c1 >= 2.0, w3, w2))
    wv3 = jnp.where((c1 >= 2.0) | (c2 >= 2.0), 0.0, w3)

    zero = jnp.zeros_like(dsel)
    a = jnp.where(eq1, w1, zero)
    a = jnp.where(eq2, wv2, a)
    a = jnp.where(cols == j3, wv3, a)

    fi = jnp.dot(a, feats_ref[...], preferred_element_type=jnp.float32)

    x = fi + fi
    mu = jnp.mean(x, axis=1, keepdims=True)
    var = jnp.mean(x * x, axis=1, keepdims=True) - mu * mu
    y = (x - mu) * jax.lax.rsqrt(var + 1e-5)
    y = y * lnw_ref[...] + lnb_ref[...]
    out_ref[...] = jnp.tanh(y) * CLAMP


@functools.partial(jax.jit, static_argnames=("interpret",))
def kernel(coords, feats, gt_coords, ln_weight, ln_bias, interpret=False):
    n = coords.shape[0]
    m = gt_coords.shape[0]
    c = feats.shape[1]

    q = gt_coords.astype(jnp.float32)
    s = coords.astype(jnp.float32)
    all_cs = jnp.concatenate([q, s], axis=0)
    mu = all_cs.mean(axis=0)
    sd = all_cs.std(axis=0, ddof=1)
    sd = jnp.where(jnp.abs(sd) < EPS, 1.0, sd)
    q = (q - mu) / sd
    s = (s - mu) / sd

    q2 = jnp.sum(q * q, axis=1, keepdims=True)
    s2 = jnp.sum(s * s, axis=1, keepdims=True)
    zq = jnp.zeros((m, 4), jnp.float32)
    qp = jnp.concatenate([-2.0 * q, zq[:, :1], q2, zq[:, :3]], axis=1)
    stp = jnp.pad(s, ((0, 0), (0, 5))).T
    ramp = (jnp.arange(1, n + 1, dtype=jnp.float32) * 1e-30).reshape(1, n)

    grid = m // QB
    fo = pl.pallas_call(
        _knn_block,
        grid=(grid,),
        in_specs=[
            pl.BlockSpec((QB, 8), lambda i: (i, 0)),
            pl.BlockSpec((8, n), lambda i: (0, 0)),
            pl.BlockSpec((1, n), lambda i: (0, 0)),
            pl.BlockSpec((1, n), lambda i: (0, 0)),
            pl.BlockSpec((n, c), lambda i: (0, 0)),
            pl.BlockSpec((1, c), lambda i: (0, 0)),
            pl.BlockSpec((1, c), lambda i: (0, 0)),
        ],
        out_specs=pl.BlockSpec((QB, c), lambda i: (i, 0)),
        out_shape=jax.ShapeDtypeStruct((m, c), jnp.float32),
        interpret=interpret,
    )(qp, stp, s2.T, ramp, feats, ln_weight.reshape(1, c), ln_bias.reshape(1, c))

    return (gt_coords, fo)
